# Initial kernel scaffold; baseline (speedup 1.0000x reference)
import functools

import jax
import jax.numpy as jnp
from jax import lax
from jax.experimental import pallas as pl
from jax.experimental.pallas import tpu as pltpu
from jax.experimental.pallas import tpu_sc as plsc

_N = 10000
_E = 320000
_D = 128
_G = 64
_EPS = 1e-5

_NC = 2
_NS = 16
_NW = _NC * _NS
_EPW = _E // _NW
_CH = 80
_NCHUNK = _EPW // _CH
_RPT = _N // _NS

_mesh = lambda: plsc.VectorSubcoreMesh(core_axis_name="c", subcore_axis_name="s")


@functools.partial(
    pl.kernel,
    out_type=jax.ShapeDtypeStruct((_NC, _N, 16), jnp.float32),
    mesh=_mesh(),
    scratch_types=[
        pltpu.VMEM((_NCHUNK, _CH), jnp.int32),
        pltpu.VMEM((_CH, 16), jnp.float32),
        pltpu.VMEM_SHARED((_N, 16), jnp.float32),
    ],
)
def _deg_kernel(dst_hbm, ones_hbm, zeros_hbm, out_hbm, dst_v, ones_v, acc):
    c = lax.axis_index("c")
    s = lax.axis_index("s")
    wid = c * _NS + s
    pltpu.sync_copy(zeros_hbm, acc.at[pl.ds(s * _RPT, _RPT)])
    pltpu.sync_copy(dst_hbm.at[wid], dst_v)
    pltpu.sync_copy(ones_hbm, ones_v)
    plsc.subcore_barrier()

    def body(j, carry):
        pltpu.sync_copy(ones_v, acc.at[dst_v.at[j]], add=True)
        return carry

    lax.fori_loop(0, _NCHUNK, body, 0)
    plsc.subcore_barrier()
    pltpu.sync_copy(acc.at[pl.ds(s * _RPT, _RPT)],
                    out_hbm.at[c, pl.ds(s * _RPT, _RPT)])


@functools.partial(
    pl.kernel,
    out_type=jax.ShapeDtypeStruct((_NC, _N, _D), jnp.float32),
    mesh=_mesh(),
    scratch_types=[
        pltpu.VMEM((_NCHUNK, _CH), jnp.int32),
        pltpu.VMEM((_NCHUNK, _CH), jnp.int32),
        pltpu.VMEM((_CH, _D), jnp.float32),
        pltpu.VMEM((_CH, _D), jnp.float32),
        pltpu.VMEM_SHARED((_N, _D), jnp.float32),
        pltpu.SemaphoreType.DMA,
        pltpu.SemaphoreType.DMA,
    ],
)
def _segsum_kernel(hp_hbm, src_hbm, dst_hbm, zeros_hbm, out_hbm,
                   src_v, dst_v, rows_a, rows_b, acc, sem_a, sem_b):
    c = lax.axis_index("c")
    s = lax.axis_index("s")
    wid = c * _NS + s
    pltpu.sync_copy(zeros_hbm, acc.at[pl.ds(s * _RPT, _RPT)])
    pltpu.sync_copy(src_hbm.at[wid], src_v)
    pltpu.sync_copy(dst_hbm.at[wid], dst_v)
    plsc.subcore_barrier()

    def gather(j, buf, sem):
        return pltpu.make_async_copy(hp_hbm.at[src_v.at[j]], buf, sem)

    gather(0, rows_a, sem_a).start()

    def body(p, carry):
        ja = 2 * p
        gather(ja, rows_a, sem_a).wait()
        gather(ja + 1, rows_b, sem_b).start()
        pltpu.sync_copy(rows_a, acc.at[dst_v.at[ja]], add=True)
        gather(ja + 1, rows_b, sem_b).wait()
        gather(ja + 2, rows_a, sem_a).start()
        pltpu.sync_copy(rows_b, acc.at[dst_v.at[ja + 1]], add=True)
        return carry

    lax.fori_loop(0, (_NCHUNK - 1) // 2, body, 0)
    gather(_NCHUNK - 1, rows_a, sem_a).wait()
    pltpu.sync_copy(rows_a, acc.at[dst_v.at[_NCHUNK - 1]], add=True)

    plsc.subcore_barrier()
    pltpu.sync_copy(acc.at[pl.ds(s * _RPT, _RPT)],
                    out_hbm.at[c, pl.ds(s * _RPT, _RPT)])


def _tc1_body(x_ref, wi_ref, degp_ref, hp1_ref, dinvb_ref):
    deg = 1.0 + degp_ref[0][:, :1] + degp_ref[1][:, :1]
    dinv_b = lax.rsqrt(deg) * jnp.ones((1, _D), jnp.float32)
    m1 = jnp.dot(x_ref[...], wi_ref[...], preferred_element_type=jnp.float32)
    dinvb_ref[...] = dinv_b
    hp1_ref[...] = dinv_b * m1


def _tc2_body(s_ref, hp_ref, dinvb_ref, b_ref, w_ref, out_ref):
    conv = dinvb_ref[...] * (s_ref[0] + s_ref[1] + hp_ref[...]) + b_ref[...]
    h = jnp.maximum(conv, 0.0)
    out_ref[...] = dinvb_ref[...] * jnp.dot(
        h, w_ref[...], preferred_element_type=jnp.float32)


def _bn_cols(h, g, b):
    m = jnp.mean(h, axis=0, keepdims=True)
    v = jnp.mean((h - m) ** 2, axis=0, keepdims=True)
    return (h - m) * lax.rsqrt(v + _EPS) * g + b


def _tc3_body(s_ref, hp_ref, dinvb_ref, b_ref, g_ref, be_ref, w_ref, out_ref):
    conv = dinvb_ref[...] * (s_ref[0] + s_ref[1] + hp_ref[...]) + b_ref[...]
    h = _bn_cols(jnp.maximum(conv, 0.0), g_ref[...], be_ref[...])
    out_ref[...] = dinvb_ref[...] * jnp.dot(
        h, w_ref[...], preferred_element_type=jnp.float32)


def _tc4_body(s_ref, hp_ref, dinvb_ref, b2_ref, g2_ref, be2_ref, batch_ref,
              fw1_ref, fb1_ref, g1h_ref, b1h_ref,
              fw2_ref, fb2_ref, g2h_ref, b2h_ref,
              wout_ref, bout_ref, out_ref):
    conv = dinvb_ref[...] * (s_ref[0] + s_ref[1] + hp_ref[...]) + b2_ref[...]
    h = _bn_cols(jnp.maximum(conv, 0.0), g2_ref[...], be2_ref[...])
    gids = lax.broadcasted_iota(jnp.int32, (_G, _N), 0)
    onehot = jnp.where(gids == batch_ref[...], 1.0, 0.0)
    hg = jnp.dot(onehot, h, preferred_element_type=jnp.float32)
    z = jnp.maximum(jnp.dot(hg, fw1_ref[...],
                            preferred_element_type=jnp.float32) + fb1_ref[...], 0.0)
    z = _bn_cols(z, g1h_ref[...], b1h_ref[...])
    z = jnp.maximum(jnp.dot(z, fw2_ref[...],
                            preferred_element_type=jnp.float32) + fb2_ref[...], 0.0)
    z = _bn_cols(z, g2h_ref[...], b2h_ref[...])
    out_ref[...] = jnp.dot(z, wout_ref[...],
                           preferred_element_type=jnp.float32) + bout_ref[...]


def _sds(shape):
    return jax.ShapeDtypeStruct(shape, jnp.float32)


def kernel(x, edge_index, batch, Wi, bi, W1, b1, g1, be1, W2, b2, g2, be2,
           fcW1, fcb1, bn1g, bn1b, fcW2, fcb2, bn2g, bn2b, Wout, bout):
    src3 = edge_index[0].reshape(_NW, _NCHUNK, _CH)
    dst3 = edge_index[1].reshape(_NW, _NCHUNK, _CH)
    ones16 = jnp.ones((_CH, 16), jnp.float32)
    zeros16 = jnp.zeros((_RPT, 16), jnp.float32)
    zerosD = jnp.zeros((_RPT, _D), jnp.float32)
    r = lambda v: v.reshape(1, -1)

    degp = _deg_kernel(dst3, ones16, zeros16)
    hp1, dinvb = pl.pallas_call(
        _tc1_body, out_shape=[_sds((_N, _D)), _sds((_N, _D))])(x, Wi, degp)

    s1 = _segsum_kernel(hp1, src3, dst3, zerosD)
    hp2 = pl.pallas_call(_tc2_body, out_shape=_sds((_N, _D)))(
        s1, hp1, dinvb, r(bi), W1)

    s2 = _segsum_kernel(hp2, src3, dst3, zerosD)
    hp3 = pl.pallas_call(_tc3_body, out_shape=_sds((_N, _D)))(
        s2, hp2, dinvb, r(b1), r(g1), r(be1), W2)

    s3 = _segsum_kernel(hp3, src3, dst3, zerosD)
    woutp = jnp.concatenate([Wout, jnp.zeros((Wout.shape[0], 7), jnp.float32)], axis=1)
    boutp = jnp.concatenate([bout, jnp.zeros((7,), jnp.float32)]).reshape(1, 8)
    out8 = pl.pallas_call(_tc4_body, out_shape=_sds((_G, 8)))(
        s3, hp3, dinvb, r(b2), r(g2), r(be2), r(batch),
        fcW1, r(fcb1), r(bn1g), r(bn1b),
        fcW2, r(fcb2), r(bn2g), r(bn2b), woutp, boutp)
    return out8[:, :1]


# speedup vs baseline: 18.8759x; 18.8759x over previous
import functools

import jax
import jax.numpy as jnp
from jax import lax
from jax.experimental import pallas as pl
from jax.experimental.pallas import tpu as pltpu
from jax.experimental.pallas import tpu_sc as plsc

_N = 10000
_E = 320000
_D = 128
_G = 64
_EPS = 1e-5

_NC = 2
_NS = 16
_NW = _NC * _NS
_EPW = _E // _NW
_CH = 80
_NCHUNK = _EPW // _CH
_BCH = 25
_NBLK = _NCHUNK // _BCH
_NPAD = 10240
_RPT = _NPAD // _NS

_mesh = lambda: plsc.VectorSubcoreMesh(core_axis_name="c", subcore_axis_name="s")


@functools.partial(
    pl.kernel,
    out_type=jax.ShapeDtypeStruct((_NC, _NPAD, _D), jnp.float32),
    mesh=_mesh(),
    scratch_types=[
        pltpu.VMEM((_NCHUNK, _CH), jnp.int32),
        pltpu.VMEM((_CH, _D), jnp.float32),
        pltpu.VMEM_SHARED((_NPAD, _D), jnp.float32),
    ],
)
def _deg_kernel(dst_hbm, ones_hbm, zeros_hbm, out_hbm, dst_v, ones_v, acc):
    c = lax.axis_index("c")
    s = lax.axis_index("s")
    wid = c * _NS + s
    pltpu.sync_copy(zeros_hbm, acc.at[pl.ds(s * _RPT, _RPT)])
    pltpu.sync_copy(dst_hbm.at[wid], dst_v)
    pltpu.sync_copy(ones_hbm, ones_v)
    plsc.subcore_barrier()

    def body(j, carry):
        pltpu.sync_copy(ones_v, acc.at[dst_v.at[j]], add=True)
        return carry

    lax.fori_loop(0, _NCHUNK, body, 0)
    plsc.subcore_barrier()
    pltpu.sync_copy(acc.at[pl.ds(s * _RPT, _RPT)],
                    out_hbm.at[c, pl.ds(s * _RPT, _RPT)])


@functools.partial(
    pl.kernel,
    out_type=jax.ShapeDtypeStruct((_NC, _NPAD, _D), jnp.float32),
    mesh=_mesh(),
    scratch_types=[
        pltpu.VMEM((_BCH, _CH), jnp.int32),
        pltpu.VMEM((_BCH, _CH), jnp.int32),
        pltpu.VMEM((_CH, _D), jnp.float32),
        pltpu.VMEM((_CH, _D), jnp.float32),
        pltpu.VMEM_SHARED((_NPAD, _D), jnp.float32),
        pltpu.SemaphoreType.DMA,
        pltpu.SemaphoreType.DMA,
    ],
)
def _segsum_kernel(hp_hbm, src_hbm, dst_hbm, zeros_hbm, out_hbm,
                   src_v, dst_v, rows_a, rows_b, acc, sem_a, sem_b):
    c = lax.axis_index("c")
    s = lax.axis_index("s")
    wid = c * _NS + s
    pltpu.sync_copy(zeros_hbm, acc.at[pl.ds(s * _RPT, _RPT)])
    plsc.subcore_barrier()

    def gather(j, buf, sem):
        return pltpu.make_async_copy(hp_hbm.at[src_v.at[j]], buf, sem)

    def block(b, carry):
        pltpu.sync_copy(src_hbm.at[wid, b], src_v)
        pltpu.sync_copy(dst_hbm.at[wid, b], dst_v)
        gather(0, rows_a, sem_a).start()

        def body(p, carry):
            ja = 2 * p
            gather(ja, rows_a, sem_a).wait()
            gather(ja + 1, rows_b, sem_b).start()
            pltpu.sync_copy(rows_a, acc.at[dst_v.at[ja]], add=True)
            gather(ja + 1, rows_b, sem_b).wait()
            gather(ja + 2, rows_a, sem_a).start()
            pltpu.sync_copy(rows_b, acc.at[dst_v.at[ja + 1]], add=True)
            return carry

        lax.fori_loop(0, (_BCH - 1) // 2, body, 0)
        gather(_BCH - 1, rows_a, sem_a).wait()
        pltpu.sync_copy(rows_a, acc.at[dst_v.at[_BCH - 1]], add=True)
        return carry

    lax.fori_loop(0, _NBLK, block, 0)
    plsc.subcore_barrier()
    pltpu.sync_copy(acc.at[pl.ds(s * _RPT, _RPT)],
                    out_hbm.at[c, pl.ds(s * _RPT, _RPT)])


def _tc1_body(x_ref, wi_ref, dinv_ref, hp1_ref, dinvb_ref):
    dinv_b = dinv_ref[...] * jnp.ones((1, _D), jnp.float32)
    m1 = jnp.dot(x_ref[...], wi_ref[...], preferred_element_type=jnp.float32)
    dinvb_ref[...] = dinv_b
    hp1_ref[...] = dinv_b * m1


def _tc2_body(s_ref, hp_ref, dinvb_ref, b_ref, w_ref, out_ref):
    conv = dinvb_ref[...] * (s_ref[0][:_N] + s_ref[1][:_N] + hp_ref[...]) + b_ref[...]
    h = jnp.maximum(conv, 0.0)
    out_ref[...] = dinvb_ref[...] * jnp.dot(
        h, w_ref[...], preferred_element_type=jnp.float32)


def _bn_cols(h, g, b):
    m = jnp.mean(h, axis=0, keepdims=True)
    v = jnp.mean((h - m) ** 2, axis=0, keepdims=True)
    return (h - m) / jnp.sqrt(v + _EPS) * g + b


def _tc3_body(s_ref, hp_ref, dinvb_ref, b_ref, g_ref, be_ref, w_ref, out_ref):
    conv = dinvb_ref[...] * (s_ref[0][:_N] + s_ref[1][:_N] + hp_ref[...]) + b_ref[...]
    h = _bn_cols(jnp.maximum(conv, 0.0), g_ref[...], be_ref[...])
    out_ref[...] = dinvb_ref[...] * jnp.dot(
        h, w_ref[...], preferred_element_type=jnp.float32)


def _tc4_body(s_ref, hp_ref, dinvb_ref, b2_ref, g2_ref, be2_ref, batch_ref,
              fw1_ref, fb1_ref, g1h_ref, b1h_ref,
              fw2_ref, fb2_ref, g2h_ref, b2h_ref,
              wout_ref, bout_ref, out_ref):
    conv = dinvb_ref[...] * (s_ref[0][:_N] + s_ref[1][:_N] + hp_ref[...]) + b2_ref[...]
    h = _bn_cols(jnp.maximum(conv, 0.0), g2_ref[...], be2_ref[...])
    gids = lax.broadcasted_iota(jnp.int32, (_G, _N), 0)
    onehot = jnp.where(gids == batch_ref[...], 1.0, 0.0)
    hg = jnp.dot(onehot, h, preferred_element_type=jnp.float32,
                 precision=lax.Precision.HIGHEST)
    z = jnp.maximum(jnp.dot(hg, fw1_ref[...],
                            preferred_element_type=jnp.float32) + fb1_ref[...], 0.0)
    z = _bn_cols(z, g1h_ref[...], b1h_ref[...])
    z = jnp.maximum(jnp.dot(z, fw2_ref[...],
                            preferred_element_type=jnp.float32) + fb2_ref[...], 0.0)
    z = _bn_cols(z, g2h_ref[...], b2h_ref[...])
    out_ref[...] = jnp.dot(z, wout_ref[...],
                           preferred_element_type=jnp.float32) + bout_ref[...]


def _sds(shape):
    return jax.ShapeDtypeStruct(shape, jnp.float32)


def kernel(x, edge_index, batch, Wi, bi, W1, b1, g1, be1, W2, b2, g2, be2,
           fcW1, fcb1, bn1g, bn1b, fcW2, fcb2, bn2g, bn2b, Wout, bout):
    src3 = edge_index[0].reshape(_NW, _NBLK, _BCH, _CH)
    dst3 = edge_index[1].reshape(_NW, _NBLK, _BCH, _CH)
    dstd = edge_index[1].reshape(_NW, _NCHUNK, _CH)
    onesD = jnp.ones((_CH, _D), jnp.float32)
    zerosD = jnp.zeros((_RPT, _D), jnp.float32)
    r = lambda v: v.reshape(1, -1)

    degp = _deg_kernel(dstd, onesD, zerosD)
    deg = 1.0 + degp[0, :_N, 0] + degp[1, :_N, 0]
    dinv = (deg ** -0.5).reshape(_N, 1)
    hp1, dinvb = pl.pallas_call(
        _tc1_body, out_shape=[_sds((_N, _D)), _sds((_N, _D))])(x, Wi, dinv)

    s1 = _segsum_kernel(hp1, src3, dst3, zerosD)
    hp2 = pl.pallas_call(_tc2_body, out_shape=_sds((_N, _D)))(
        s1, hp1, dinvb, r(bi), W1)

    s2 = _segsum_kernel(hp2, src3, dst3, zerosD)
    hp3 = pl.pallas_call(_tc3_body, out_shape=_sds((_N, _D)))(
        s2, hp2, dinvb, r(b1), r(g1), r(be1), W2)

    s3 = _segsum_kernel(hp3, src3, dst3, zerosD)
    woutp = jnp.concatenate([Wout, jnp.zeros((Wout.shape[0], 7), jnp.float32)], axis=1)
    boutp = jnp.concatenate([bout, jnp.zeros((7,), jnp.float32)]).reshape(1, 8)
    out8 = pl.pallas_call(_tc4_body, out_shape=_sds((_G, 8)))(
        s3, hp3, dinvb, r(b2), r(g2), r(be2), r(batch),
        fcW1, r(fcb1), r(bn1g), r(bn1b),
        fcW2, r(fcb2), r(bn2g), r(bn2b), woutp, boutp)
    return out8[:, :1]


# speedup vs baseline: 18.9921x; 1.0062x over previous
import functools

import jax
import jax.numpy as jnp
from jax import lax
from jax.experimental import pallas as pl
from jax.experimental.pallas import tpu as pltpu
from jax.experimental.pallas import tpu_sc as plsc

_N = 10000
_E = 320000
_D = 128
_G = 64
_EPS = 1e-5

_NC = 2
_NS = 16
_NW = _NC * _NS
_EPW = _E // _NW
_CH = 80
_NCHUNK = _EPW // _CH
_BCH = 25
_NBLK = _NCHUNK // _BCH
_NPAD = 10240
_RPT = _NPAD // _NS

_mesh = lambda: plsc.VectorSubcoreMesh(core_axis_name="c", subcore_axis_name="s")


@functools.partial(
    pl.kernel,
    out_type=jax.ShapeDtypeStruct((_NC, _NPAD, _D), jnp.float32),
    mesh=_mesh(),
    scratch_types=[
        pltpu.VMEM((_NCHUNK, _CH), jnp.int32),
        pltpu.VMEM((_CH, _D), jnp.float32),
        pltpu.VMEM_SHARED((_NPAD, _D), jnp.float32),
        pltpu.SemaphoreType.DMA,
    ],
)
def _deg_kernel(dst_hbm, ones_hbm, zeros_hbm, out_hbm, dst_v, ones_v, acc,
                deg_sem):
    c = lax.axis_index("c")
    s = lax.axis_index("s")
    wid = c * _NS + s
    pltpu.sync_copy(zeros_hbm, acc.at[pl.ds(s * _RPT, _RPT)])
    pltpu.sync_copy(dst_hbm.at[wid], dst_v)
    pltpu.sync_copy(ones_hbm, ones_v)
    plsc.subcore_barrier()

    def scat(j):
        return pltpu.async_copy(ones_v, acc.at[dst_v.at[j]], deg_sem, add=True)

    def scatw(j):
        pltpu.make_async_copy(ones_v, acc.at[dst_v.at[j]], deg_sem).wait()

    for k in range(4):
        scat(k)

    def body(j, carry):
        scatw(j - 4)
        scat(j)
        return carry

    lax.fori_loop(4, _NCHUNK, body, 0)
    for k in range(4):
        scatw(_NCHUNK - 4 + k)
    plsc.subcore_barrier()
    pltpu.sync_copy(acc.at[pl.ds(s * _RPT, _RPT)],
                    out_hbm.at[c, pl.ds(s * _RPT, _RPT)])


@functools.partial(
    pl.kernel,
    out_type=jax.ShapeDtypeStruct((_NC, _NPAD, _D), jnp.float32),
    mesh=_mesh(),
    scratch_types=[
        pltpu.VMEM((_BCH, _CH), jnp.int32),
        pltpu.VMEM((_BCH, _CH), jnp.int32),
        pltpu.VMEM((_CH, _D), jnp.float32),
        pltpu.VMEM((_CH, _D), jnp.float32),
        pltpu.VMEM_SHARED((_NPAD, _D), jnp.float32),
        pltpu.SemaphoreType.DMA,
        pltpu.SemaphoreType.DMA,
        pltpu.SemaphoreType.DMA,
        pltpu.SemaphoreType.DMA,
    ],
)
def _segsum_kernel(hp_hbm, src_hbm, dst_hbm, zeros_hbm, out_hbm,
                   src_v, dst_v, rows_a, rows_b, acc,
                   sga, sgb, ssa, ssb):
    c = lax.axis_index("c")
    s = lax.axis_index("s")
    wid = c * _NS + s
    pltpu.sync_copy(zeros_hbm, acc.at[pl.ds(s * _RPT, _RPT)])
    plsc.subcore_barrier()

    def gather(j, buf, sem):
        return pltpu.make_async_copy(hp_hbm.at[src_v.at[j]], buf, sem)

    def scat(j, buf, sem):
        pltpu.async_copy(buf, acc.at[dst_v.at[j]], sem, add=True)

    def scatw(j, buf, sem):
        pltpu.make_async_copy(buf, acc.at[dst_v.at[j]], sem).wait()

    def block(b, carry):
        pltpu.sync_copy(src_hbm.at[wid, b], src_v)
        pltpu.sync_copy(dst_hbm.at[wid, b], dst_v)
        gather(0, rows_a, sga).start()
        gather(1, rows_b, sgb).start()
        gather(0, rows_a, sga).wait()
        scat(0, rows_a, ssa)
        gather(1, rows_b, sgb).wait()
        scat(1, rows_b, ssb)

        def body(p, carry):
            ja = 2 * p
            scatw(ja - 2, rows_a, ssa)
            gather(ja, rows_a, sga).start()
            scatw(ja - 1, rows_b, ssb)
            gather(ja + 1, rows_b, sgb).start()
            gather(ja, rows_a, sga).wait()
            scat(ja, rows_a, ssa)
            gather(ja + 1, rows_b, sgb).wait()
            scat(ja + 1, rows_b, ssb)
            return carry

        lax.fori_loop(1, (_BCH - 1) // 2, body, 0)
        scatw(_BCH - 3, rows_a, ssa)
        gather(_BCH - 1, rows_a, sga).start()
        gather(_BCH - 1, rows_a, sga).wait()
        scat(_BCH - 1, rows_a, ssa)
        scatw(_BCH - 2, rows_b, ssb)
        scatw(_BCH - 1, rows_a, ssa)
        return carry

    lax.fori_loop(0, _NBLK, block, 0)
    plsc.subcore_barrier()
    pltpu.sync_copy(acc.at[pl.ds(s * _RPT, _RPT)],
                    out_hbm.at[c, pl.ds(s * _RPT, _RPT)])


def _tc1_body(x_ref, wi_ref, dinv_ref, hp1_ref, dinvb_ref):
    dinv_b = dinv_ref[...] * jnp.ones((1, _D), jnp.float32)
    m1 = jnp.dot(x_ref[...], wi_ref[...], preferred_element_type=jnp.float32)
    dinvb_ref[...] = dinv_b
    hp1_ref[...] = dinv_b * m1


def _tc2_body(s_ref, hp_ref, dinvb_ref, b_ref, w_ref, out_ref):
    conv = dinvb_ref[...] * (s_ref[0][:_N] + s_ref[1][:_N] + hp_ref[...]) + b_ref[...]
    h = jnp.maximum(conv, 0.0)
    out_ref[...] = dinvb_ref[...] * jnp.dot(
        h, w_ref[...], preferred_element_type=jnp.float32)


def _bn_cols(h, g, b):
    m = jnp.mean(h, axis=0, keepdims=True)
    v = jnp.mean((h - m) ** 2, axis=0, keepdims=True)
    return (h - m) / jnp.sqrt(v + _EPS) * g + b


def _tc3_body(s_ref, hp_ref, dinvb_ref, b_ref, g_ref, be_ref, w_ref, out_ref):
    conv = dinvb_ref[...] * (s_ref[0][:_N] + s_ref[1][:_N] + hp_ref[...]) + b_ref[...]
    h = _bn_cols(jnp.maximum(conv, 0.0), g_ref[...], be_ref[...])
    out_ref[...] = dinvb_ref[...] * jnp.dot(
        h, w_ref[...], preferred_element_type=jnp.float32)


def _tc4_body(s_ref, hp_ref, dinvb_ref, b2_ref, g2_ref, be2_ref, batch_ref,
              fw1_ref, fb1_ref, g1h_ref, b1h_ref,
              fw2_ref, fb2_ref, g2h_ref, b2h_ref,
              wout_ref, bout_ref, out_ref):
    conv = dinvb_ref[...] * (s_ref[0][:_N] + s_ref[1][:_N] + hp_ref[...]) + b2_ref[...]
    h = _bn_cols(jnp.maximum(conv, 0.0), g2_ref[...], be2_ref[...])
    gids = lax.broadcasted_iota(jnp.int32, (_G, _N), 0)
    onehot = jnp.where(gids == batch_ref[...], 1.0, 0.0)
    hg = jnp.dot(onehot, h, preferred_element_type=jnp.float32,
                 precision=lax.Precision.HIGHEST)
    z = jnp.maximum(jnp.dot(hg, fw1_ref[...],
                            preferred_element_type=jnp.float32) + fb1_ref[...], 0.0)
    z = _bn_cols(z, g1h_ref[...], b1h_ref[...])
    z = jnp.maximum(jnp.dot(z, fw2_ref[...],
                            preferred_element_type=jnp.float32) + fb2_ref[...], 0.0)
    z = _bn_cols(z, g2h_ref[...], b2h_ref[...])
    out_ref[...] = jnp.dot(z, wout_ref[...],
                           preferred_element_type=jnp.float32) + bout_ref[...]


def _sds(shape):
    return jax.ShapeDtypeStruct(shape, jnp.float32)


def kernel(x, edge_index, batch, Wi, bi, W1, b1, g1, be1, W2, b2, g2, be2,
           fcW1, fcb1, bn1g, bn1b, fcW2, fcb2, bn2g, bn2b, Wout, bout):
    src3 = edge_index[0].reshape(_NW, _NBLK, _BCH, _CH)
    dst3 = edge_index[1].reshape(_NW, _NBLK, _BCH, _CH)
    dstd = edge_index[1].reshape(_NW, _NCHUNK, _CH)
    onesD = jnp.ones((_CH, _D), jnp.float32)
    zerosD = jnp.zeros((_RPT, _D), jnp.float32)
    r = lambda v: v.reshape(1, -1)

    degp = _deg_kernel(dstd, onesD, zerosD)
    deg = 1.0 + degp[0, :_N, 0] + degp[1, :_N, 0]
    dinv = (deg ** -0.5).reshape(_N, 1)
    hp1, dinvb = pl.pallas_call(
        _tc1_body, out_shape=[_sds((_N, _D)), _sds((_N, _D))])(x, Wi, dinv)

    s1 = _segsum_kernel(hp1, src3, dst3, zerosD)
    hp2 = pl.pallas_call(_tc2_body, out_shape=_sds((_N, _D)))(
        s1, hp1, dinvb, r(bi), W1)

    s2 = _segsum_kernel(hp2, src3, dst3, zerosD)
    hp3 = pl.pallas_call(_tc3_body, out_shape=_sds((_N, _D)))(
        s2, hp2, dinvb, r(b1), r(g1), r(be1), W2)

    s3 = _segsum_kernel(hp3, src3, dst3, zerosD)
    woutp = jnp.concatenate([Wout, jnp.zeros((Wout.shape[0], 7), jnp.float32)], axis=1)
    boutp = jnp.concatenate([bout, jnp.zeros((7,), jnp.float32)]).reshape(1, 8)
    out8 = pl.pallas_call(_tc4_body, out_shape=_sds((_G, 8)))(
        s3, hp3, dinvb, r(b2), r(g2), r(be2), r(batch),
        fcW1, r(fcb1), r(bn1g), r(bn1b),
        fcW2, r(fcb2), r(bn2g), r(bn2b), woutp, boutp)
    return out8[:, :1]
